# baseline (speedup 1.0000x reference)
import functools

import numpy as np
import jax
import jax.numpy as jnp
from jax import lax
from jax.experimental import pallas as pl
from jax.experimental.pallas import tpu as pltpu
from jax.experimental.pallas import tpu_sc as plsc

_D = 128
_MAX_LEN = 1000
_B = 1024
_L = 128
_NC = 2
_NS = 16
_NW = _NC * _NS
_N = _B * _L
_PER_W = _N // _NW
_CH = 128
_NCH = _PER_W // _CH
_NB = 6


def _make_pe():
    position = np.arange(_MAX_LEN, dtype=np.float32)[:, None]
    div_term = np.exp(
        np.arange(0, _D, 2, dtype=np.float32) * (-np.log(10000.0) / _D))
    pe = np.zeros((_MAX_LEN, _D), dtype=np.float32)
    pe[:, 0::2] = np.sin(position * div_term)
    pe[:, 1::2] = np.cos(position * div_term)
    return jnp.asarray(pe[:_L])


_mesh = plsc.VectorSubcoreMesh(
    core_axis_name="c", subcore_axis_name="s",
    num_cores=_NC, num_subcores=_NS)


@functools.partial(
    pl.kernel,
    out_type=jax.ShapeDtypeStruct((_N, _D), jnp.float32),
    mesh=_mesh,
    scratch_types=(
        [pltpu.VMEM((_NCH, _CH), jnp.int32),
         pltpu.VMEM((_L, _D), jnp.float32)]
        + [pltpu.VMEM((_CH, _D), jnp.float32) for _ in range(_NB)]
        + [pltpu.SemaphoreType.DMA for _ in range(2 * _NB)]
    ),
)
def _emb_kernel(w_hbm, xr_hbm, pe_hbm, out_hbm, idx_v, pe_v, *sc):
    bufs = sc[:_NB]
    gsems = sc[_NB:2 * _NB]
    osems = sc[2 * _NB:]
    wid = lax.axis_index("s") * _NC + lax.axis_index("c")
    base = wid * _PER_W
    pltpu.sync_copy(xr_hbm.at[wid], idx_v)
    pltpu.sync_copy(pe_hbm, pe_v)

    for b in range(_NB):
        pltpu.async_copy(w_hbm.at[idx_v.at[b]], bufs[b], gsems[b])

    def add_pe(buf):
        def rows(r2, c_):
            r = r2 * 2
            for rr in (r, r + 1):
                for c in range(8):
                    s = pl.ds(c * 16, 16)
                    buf[rr, s] = buf[rr, s] + pe_v[rr, s]
            return c_
        lax.fori_loop(0, _CH // 2, rows, 0)

    for i in range(_NCH):
        b = i % _NB
        pltpu.make_async_copy(
            w_hbm.at[idx_v.at[i]], bufs[b], gsems[b]).wait()
        add_pe(bufs[b])
        pltpu.async_copy(
            bufs[b], out_hbm.at[pl.ds(base + i * _CH, _CH)], osems[b])
        j = i - 1 + _NB
        if i >= 1 and j < _NCH:
            pb = (i - 1) % _NB
            pltpu.make_async_copy(
                bufs[pb], out_hbm.at[pl.ds(base, _CH)], osems[pb]).wait()
            pltpu.async_copy(w_hbm.at[idx_v.at[j]], bufs[pb], gsems[pb])

    for b in range(_NB):
        pltpu.make_async_copy(
            bufs[b], out_hbm.at[pl.ds(base, _CH)], osems[b]).wait()


def kernel(x, W):
    pe = _make_pe()
    xr = x.reshape(_NW, _NCH, _CH)
    out = _emb_kernel(W, xr, pe)
    return out.reshape(_B, _L, _D)


# speedup vs baseline: 1.0677x; 1.0677x over previous
import functools

import numpy as np
import jax
import jax.numpy as jnp
from jax import lax
from jax.experimental import pallas as pl
from jax.experimental.pallas import tpu as pltpu
from jax.experimental.pallas import tpu_sc as plsc

_D = 128
_MAX_LEN = 1000
_B = 1024
_L = 128
_NC = 2
_NS = 16
_NW = _NC * _NS
_N = _B * _L
_PER_W = _N // _NW
_CH = 128
_NCH = _PER_W // _CH
_NGB = 4
_NOB = 2
_NG = _NCH // _NGB


def _make_pe():
    position = np.arange(_MAX_LEN, dtype=np.float32)[:, None]
    div_term = np.exp(
        np.arange(0, _D, 2, dtype=np.float32) * (-np.log(10000.0) / _D))
    pe = np.zeros((_MAX_LEN, _D), dtype=np.float32)
    pe[:, 0::2] = np.sin(position * div_term)
    pe[:, 1::2] = np.cos(position * div_term)
    return jnp.asarray(pe[:_L])


_mesh = plsc.VectorSubcoreMesh(
    core_axis_name="c", subcore_axis_name="s",
    num_cores=_NC, num_subcores=_NS)


@functools.partial(
    pl.kernel,
    out_type=jax.ShapeDtypeStruct((_N, _D), jnp.float32),
    mesh=_mesh,
    scratch_types=(
        [pltpu.VMEM((_NCH, _CH), jnp.int32),
         pltpu.VMEM((_L, _D), jnp.float32)]
        + [pltpu.VMEM((_CH, _D), jnp.float32) for _ in range(_NGB + _NOB)]
        + [pltpu.SemaphoreType.DMA for _ in range(_NGB + _NOB)]
    ),
)
def _emb_kernel(w_hbm, xr_hbm, pe_hbm, out_hbm, idx_v, pe_v, *sc):
    gbufs = sc[:_NGB]
    obufs = sc[_NGB:_NGB + _NOB]
    gsems = sc[_NGB + _NOB:2 * _NGB + _NOB]
    osems = sc[2 * _NGB + _NOB:]
    wid = lax.axis_index("s") * _NC + lax.axis_index("c")
    base = wid * _PER_W
    pltpu.sync_copy(xr_hbm.at[wid], idx_v)
    pltpu.sync_copy(pe_hbm, pe_v)

    for b in range(_NGB):
        pltpu.async_copy(w_hbm.at[idx_v.at[b]], gbufs[b], gsems[b])

    def add_pe(src, dst):
        def rows(r2, c_):
            r = r2 * 2
            for rr in (r, r + 1):
                for c in range(8):
                    s = pl.ds(c * 16, 16)
                    dst[rr, s] = src[rr, s] + pe_v[rr, s]
            return c_
        lax.fori_loop(0, _CH // 2, rows, 0)

    def group(g, carry):
        a0 = g * _NGB
        for b in range(_NGB):
            ob = b % _NOB
            pltpu.make_async_copy(
                w_hbm.at[idx_v.at[a0 + b]], gbufs[b], gsems[b]).wait()
            if b >= _NOB:
                pltpu.make_async_copy(
                    obufs[ob], out_hbm.at[pl.ds(base, _CH)], osems[ob]).wait()
            else:
                @pl.when(g > 0)
                def _(ob=ob):
                    pltpu.make_async_copy(
                        obufs[ob], out_hbm.at[pl.ds(base, _CH)],
                        osems[ob]).wait()
            add_pe(gbufs[b], obufs[ob])

            @pl.when(g < _NG - 1)
            def _(b=b):
                pltpu.async_copy(
                    w_hbm.at[idx_v.at[a0 + _NGB + b]], gbufs[b], gsems[b])
            pltpu.async_copy(
                obufs[ob], out_hbm.at[pl.ds(base + (a0 + b) * _CH, _CH)],
                osems[ob])
        return carry

    lax.fori_loop(0, _NG, group, 0)
    for ob in range(_NOB):
        pltpu.make_async_copy(
            obufs[ob], out_hbm.at[pl.ds(base, _CH)], osems[ob]).wait()


def kernel(x, W):
    pe = _make_pe()
    xr = x.reshape(_NW, _NCH, _CH)
    out = _emb_kernel(W, xr, pe)
    return out.reshape(_B, _L, _D)
